# baseline (speedup 1.0000x reference)
import functools

import jax
import jax.numpy as jnp
from jax import lax
from jax.experimental import pallas as pl
from jax.experimental.pallas import tpu as pltpu
from jax.experimental.pallas import tpu_sc as plsc

_BATCH = 16384
_FEAT_DIM = 64
_L = 16
_BLK = 128
_FG = 16
_NPASS = _FEAT_DIM // _FG

_info = plsc.get_sparse_core_info()
_NC, _NS = _info.num_cores, _info.num_subcores
_NW = _NC * _NS
_B_PER_W = _BATCH // _NW
_NGROUP = _B_PER_W // _L
_RING = 32
_FCHUNK = 16

_mesh = plsc.VectorSubcoreMesh(core_axis_name="c", subcore_axis_name="s")


@functools.partial(
    pl.kernel,
    mesh=_mesh,
    out_type=jax.ShapeDtypeStruct((_NW, _L), jnp.float32),
    scratch_types=[
        pltpu.VMEM((_B_PER_W,), jnp.int32),
        pltpu.VMEM((_B_PER_W,), jnp.int32),
        pltpu.VMEM((_B_PER_W,), jnp.int32),
        pltpu.VMEM((_B_PER_W // 2, 2 * _FEAT_DIM), jnp.float32),
        pltpu.VMEM((_RING, _FG, _BLK), jnp.float32),
        pltpu.VMEM((_L,), jnp.float32),
        pltpu.SemaphoreType.DMA,
        pltpu.SemaphoreType.DMA,
    ],
    compiler_params=pltpu.CompilerParams(needs_layout_passes=False),
)
def _center_loss_partials(featsP_hbm, slots_hbm, cols_hbm, fetch_hbm,
                          centersT_hbm, out_hbm,
                          slots_v, cols_v, fetch_v, feats_v, ring_v, acc_v,
                          sem, fsem):
    wid = lax.axis_index("s") * _NC + lax.axis_index("c")

    pltpu.sync_copy(slots_hbm.at[wid], slots_v)
    pltpu.sync_copy(cols_hbm.at[wid], cols_v)
    pltpu.sync_copy(fetch_hbm.at[wid], fetch_v)
    fcp = pltpu.async_copy(featsP_hbm.at[wid], feats_v, fsem)

    lanes = lax.iota(jnp.int32, _L)
    drain_src = centersT_hbm.at[pl.ds(0, _FG), pl.ds(0, _BLK)]

    fcp.wait()

    acc = jnp.zeros((_L,), jnp.float32)
    for p in range(_NPASS):
        frow = pl.ds(p * _FG, _FG)

        def fire_chunk(c, _, frow=frow):
            bidv = fetch_v[pl.ds(c * _FCHUNK, _FCHUNK)]
            for k in range(_FCHUNK):
                off = pl.multiple_of(bidv[k] * _BLK, _BLK)
                pltpu.async_copy(
                    centersT_hbm.at[frow, pl.ds(off, _BLK)],
                    ring_v.at[(c * _FCHUNK + k) & (_RING - 1)], sem)
            for k in range(_FCHUNK):
                pltpu.make_async_copy(
                    drain_src, ring_v.at[0], sem).wait()
            return 0

        def group(g, carry, p=p):
            a, fired = carry
            slotv = slots_v[pl.ds(g * _L, _L)]
            needed = (slotv[_L - 1] >> 4) + 1
            lax.fori_loop(fired, needed, fire_chunk, 0)
            fired = needed
            colv = cols_v[pl.ds(g * _L, _L)]
            for e in range(_L):
                srm = (jnp.zeros((_L,), jnp.int32) + slotv[e]) & jnp.int32(_RING - 1)
                cvec = jnp.zeros((_L,), jnp.int32) + colv[e]
                cv = plsc.load_gather(ring_v, [srm, lanes, cvec])
                fb = (e & 1) * _FEAT_DIM + p * _FG
                fv = feats_v[g * (_L // 2) + (e >> 1), pl.ds(fb, _L)]
                d = fv - cv
                a = a + d * d
            return a, fired

        acc, _ = lax.fori_loop(0, _NGROUP, group, (acc, jnp.int32(0)))

    acc_v[...] = acc
    pltpu.sync_copy(acc_v, out_hbm.at[wid])


def kernel(feats, labels, centers):
    labels_i32 = labels.astype(jnp.int32)
    bid_full = labels_i32 >> 7
    perm = jnp.argsort(bid_full)
    sl = labels_i32[perm]
    feats_s = feats[perm]

    bid2 = (sl >> 7).reshape(_NW, _B_PER_W)
    col2 = (sl & jnp.int32(_BLK - 1)).reshape(_NW, _B_PER_W)
    first = jnp.ones((_NW, 1), bool)
    nf = jnp.concatenate([first, bid2[:, 1:] != bid2[:, :-1]], axis=1)
    slot2 = jnp.cumsum(nf.astype(jnp.int32), axis=1) - 1
    rows = jax.lax.broadcasted_iota(jnp.int32, (_NW, _B_PER_W), 0)
    dst = jnp.where(nf, slot2, _B_PER_W)
    fetch2 = jnp.zeros((_NW, _B_PER_W + 1), jnp.int32).at[rows, dst].set(
        bid2, mode="drop")[:, :_B_PER_W]
    featsP = feats_s.reshape(_NW, _B_PER_W // 2, 2 * _FEAT_DIM)

    partials = _center_loss_partials(featsP, slot2, col2, fetch2, centers.T)
    return jnp.sum(partials) / (2.0 * _BATCH)


# speedup vs baseline: 1.3528x; 1.3528x over previous
import functools

import jax
import jax.numpy as jnp
from jax import lax
from jax.experimental import pallas as pl
from jax.experimental.pallas import tpu as pltpu
from jax.experimental.pallas import tpu_sc as plsc

_BATCH = 16384
_FEAT_DIM = 64
_L = 16
_BLK = 128
_FG = 16
_NPASS = _FEAT_DIM // _FG

_info = plsc.get_sparse_core_info()
_NC, _NS = _info.num_cores, _info.num_subcores
_NW = _NC * _NS
_B_PER_W = _BATCH // _NW
_NGROUP = _B_PER_W // _L
_RING = 32
_FCHUNK = 16

_mesh = plsc.VectorSubcoreMesh(core_axis_name="c", subcore_axis_name="s")


@functools.partial(
    pl.kernel,
    mesh=_mesh,
    out_type=jax.ShapeDtypeStruct((_NW, _L), jnp.float32),
    scratch_types=[
        pltpu.VMEM((_B_PER_W,), jnp.int32),
        pltpu.VMEM((_B_PER_W,), jnp.int32),
        pltpu.VMEM((_B_PER_W,), jnp.int32),
        pltpu.VMEM((_B_PER_W // 2, 2 * _FEAT_DIM), jnp.float32),
        pltpu.VMEM((_RING, _FG, _BLK), jnp.float32),
        pltpu.VMEM((_L,), jnp.float32),
        pltpu.SemaphoreType.DMA,
        pltpu.SemaphoreType.DMA,
    ],
    compiler_params=pltpu.CompilerParams(needs_layout_passes=False),
)
def _center_loss_partials(featsP_hbm, slots_hbm, cols_hbm, fetch_hbm,
                          centersT_hbm, out_hbm,
                          slots_v, cols_v, fetch_v, feats_v, ring_v, acc_v,
                          sem, fsem):
    wid = lax.axis_index("s") * _NC + lax.axis_index("c")

    pltpu.sync_copy(slots_hbm.at[wid], slots_v)
    pltpu.sync_copy(cols_hbm.at[wid], cols_v)
    pltpu.sync_copy(fetch_hbm.at[wid], fetch_v)
    fcp = pltpu.async_copy(featsP_hbm.at[wid], feats_v, fsem)

    lanes = lax.iota(jnp.int32, _L)
    drain_src = centersT_hbm.at[pl.ds(0, _FG), pl.ds(0, _BLK)]

    fcp.wait()

    acc = jnp.zeros((_L,), jnp.float32)
    for p in range(_NPASS):
        frow = pl.ds(p * _FG, _FG)

        def fire_chunk(c, _, frow=frow):
            bidv = fetch_v[pl.ds(c * _FCHUNK, _FCHUNK)]
            for k in range(_FCHUNK):
                off = pl.multiple_of(bidv[k] * _BLK, _BLK)
                pltpu.async_copy(
                    centersT_hbm.at[frow, pl.ds(off, _BLK)],
                    ring_v.at[(c * _FCHUNK + k) & (_RING - 1)], sem)
            for k in range(_FCHUNK):
                pltpu.make_async_copy(
                    drain_src, ring_v.at[0], sem).wait()
            return 0

        def group(g, carry, p=p):
            a, fired = carry
            slotv = slots_v[pl.ds(g * _L, _L)]
            needed = (slotv[_L - 1] >> 4) + 1
            lax.fori_loop(fired, needed, fire_chunk, 0)
            fired = needed
            colv = cols_v[pl.ds(g * _L, _L)]
            for e in range(_L):
                srm = (jnp.zeros((_L,), jnp.int32) + slotv[e]) & jnp.int32(_RING - 1)
                cvec = jnp.zeros((_L,), jnp.int32) + colv[e]
                cv = plsc.load_gather(ring_v, [srm, lanes, cvec])
                fb = (e & 1) * _FEAT_DIM + p * _FG
                fv = feats_v[g * (_L // 2) + (e >> 1), pl.ds(fb, _L)]
                d = fv - cv
                a = a + d * d
            return a, fired

        acc, _ = lax.fori_loop(0, _NGROUP, group, (acc, jnp.int32(0)))

    acc_v[...] = acc
    pltpu.sync_copy(acc_v, out_hbm.at[wid])


def kernel(feats, labels, centers):
    labels_i32 = labels.astype(jnp.int32)
    bid_full = labels_i32 >> 7
    perm = jnp.argsort(bid_full)
    sl = labels_i32[perm]
    feats_s = feats[perm]

    bid2 = (sl >> 7).reshape(_NW, _B_PER_W)
    col2 = (sl & jnp.int32(_BLK - 1)).reshape(_NW, _B_PER_W)
    first = jnp.ones((_NW, 1), bool)
    nf = jnp.concatenate([first, bid2[:, 1:] != bid2[:, :-1]], axis=1)
    slot2 = jnp.cumsum(nf.astype(jnp.int32), axis=1) - 1
    order = jnp.argsort(jnp.logical_not(nf), axis=1, stable=True)
    fetch2 = jnp.take_along_axis(bid2, order, axis=1)
    featsP = feats_s.reshape(_NW, _B_PER_W // 2, 2 * _FEAT_DIM)

    partials = _center_loss_partials(featsP, slot2, col2, fetch2, centers.T)
    return jnp.sum(partials) / (2.0 * _BATCH)


# speedup vs baseline: 1.6607x; 1.2277x over previous
import functools

import jax
import jax.numpy as jnp
from jax import lax
from jax.experimental import pallas as pl
from jax.experimental.pallas import tpu as pltpu
from jax.experimental.pallas import tpu_sc as plsc

_BATCH = 16384
_FEAT_DIM = 64
_L = 16
_BLK = 128
_FG = 16
_NPASS = _FEAT_DIM // _FG

_info = plsc.get_sparse_core_info()
_NC, _NS = _info.num_cores, _info.num_subcores
_NW = _NC * _NS
_B_PER_W = _BATCH // _NW
_NGROUP = _B_PER_W // _L
_RING = 32
_FCHUNK = 8
_NFCHUNK = _B_PER_W // _FCHUNK

_mesh = plsc.VectorSubcoreMesh(core_axis_name="c", subcore_axis_name="s")


@functools.partial(
    pl.kernel,
    mesh=_mesh,
    out_type=jax.ShapeDtypeStruct((_NW, _L), jnp.float32),
    scratch_types=[
        pltpu.VMEM((_B_PER_W,), jnp.int32),
        pltpu.VMEM((_B_PER_W,), jnp.int32),
        pltpu.VMEM((_B_PER_W + _L,), jnp.int32),
        pltpu.VMEM((_B_PER_W // 2, 2 * _FEAT_DIM), jnp.float32),
        pltpu.VMEM((_RING, _FG, _BLK), jnp.float32),
        pltpu.VMEM((_L,), jnp.float32),
        pltpu.SemaphoreType.DMA,
        pltpu.SemaphoreType.DMA,
    ],
    compiler_params=pltpu.CompilerParams(needs_layout_passes=False),
)
def _center_loss_partials(featsP_hbm, slots_hbm, cols_hbm, fetch_hbm,
                          centersT_hbm, out_hbm,
                          slots_v, cols_v, fetch_v, feats_v, ring_v, acc_v,
                          sem, fsem):
    wid = lax.axis_index("s") * _NC + lax.axis_index("c")

    pltpu.sync_copy(slots_hbm.at[wid], slots_v)
    pltpu.sync_copy(cols_hbm.at[wid], cols_v)
    pltpu.sync_copy(fetch_hbm.at[wid], fetch_v)
    fcp = pltpu.async_copy(featsP_hbm.at[wid], feats_v, fsem)

    lanes = lax.iota(jnp.int32, _L)
    drain_src = centersT_hbm.at[pl.ds(0, _FG), pl.ds(0, _BLK)]

    fcp.wait()

    acc = jnp.zeros((_L,), jnp.float32)
    for p in range(_NPASS):
        frow = pl.ds(p * _FG, _FG)

        def fire_chunk(c, _, frow=frow):
            bidv = fetch_v[pl.ds(c * _FCHUNK, _L)]
            for k in range(_FCHUNK):
                off = pl.multiple_of(bidv[k] * _BLK, _BLK)
                pltpu.async_copy(
                    centersT_hbm.at[frow, pl.ds(off, _BLK)],
                    ring_v.at[(c * _FCHUNK + k) & (_RING - 1)], sem)
            return 0

        def drain_chunk(c, _):
            for k in range(_FCHUNK):
                pltpu.make_async_copy(
                    drain_src, ring_v.at[0], sem).wait()
            return 0

        def group(g, carry, p=p):
            a, fired, drained = carry
            slotv = slots_v[pl.ds(g * _L, _L)]
            needed = (slotv[_L - 1] >> 3) + 1
            fire_to = jnp.minimum(needed + 1, _NFCHUNK)
            lax.fori_loop(fired, fire_to, fire_chunk, 0)
            lax.fori_loop(drained, needed, drain_chunk, 0)
            fired = fire_to
            drained = needed
            colv = cols_v[pl.ds(g * _L, _L)]
            for e in range(_L):
                srm = (jnp.zeros((_L,), jnp.int32) + slotv[e]) & jnp.int32(_RING - 1)
                cvec = jnp.zeros((_L,), jnp.int32) + colv[e]
                cv = plsc.load_gather(ring_v, [srm, lanes, cvec])
                fb = (e & 1) * _FEAT_DIM + p * _FG
                fv = feats_v[g * (_L // 2) + (e >> 1), pl.ds(fb, _L)]
                d = fv - cv
                a = a + d * d
            return a, fired, drained

        acc, fired, drained = lax.fori_loop(
            0, _NGROUP, group, (acc, jnp.int32(0), jnp.int32(0)))
        lax.fori_loop(drained, fired, drain_chunk, 0)

    acc_v[...] = acc
    pltpu.sync_copy(acc_v, out_hbm.at[wid])


def kernel(feats, labels, centers):
    labels_i32 = labels.astype(jnp.int32)
    bid_full = labels_i32 >> 7
    perm = jnp.argsort(bid_full)
    sl = labels_i32[perm]
    feats_s = feats[perm]

    bid2 = (sl >> 7).reshape(_NW, _B_PER_W)
    col2 = (sl & jnp.int32(_BLK - 1)).reshape(_NW, _B_PER_W)
    first = jnp.ones((_NW, 1), bool)
    nf = jnp.concatenate([first, bid2[:, 1:] != bid2[:, :-1]], axis=1)
    slot2 = jnp.cumsum(nf.astype(jnp.int32), axis=1) - 1
    order = jnp.argsort(jnp.logical_not(nf), axis=1, stable=True)
    fetch2 = jnp.take_along_axis(bid2, order, axis=1)
    fetch2 = jnp.concatenate(
        [fetch2, jnp.zeros((_NW, _L), jnp.int32)], axis=1)
    featsP = feats_s.reshape(_NW, _B_PER_W // 2, 2 * _FEAT_DIM)

    partials = _center_loss_partials(featsP, slot2, col2, fetch2, centers.T)
    return jnp.sum(partials) / (2.0 * _BATCH)
